# baseline (speedup 1.0000x reference)
import functools

import jax
import jax.numpy as jnp
from jax import lax
from jax.experimental import pallas as pl
from jax.experimental.pallas import tpu as pltpu
from jax.experimental.pallas import tpu_sc as plsc

_NUM_CLASSES = 15
_THRESH = 0.05
_TOPK = 2000
_N = 384 * 384
_TOT = _N * _NUM_CLASSES

_NW = 32
_CHUNK = _TOT // _NW
_NVEC = _CHUNK // 16
_HBINS = 2048
_L2BINS = 4096
_FBINS = 2048
_CAP = 4096
_CAPROW = 4224
_BIAS = 0x3D000000
_BIG_FLAT = _TOT + 1


def _mean_body(x_ref, o_ref):
    s = jax.nn.sigmoid(x_ref[...])
    o_ref[...] = jnp.mean(s, axis=0)


def _scores_mean_cm(cls_score):
    x = cls_score.reshape(8, _NUM_CLASSES, 1152, 128)
    return pl.pallas_call(
        _mean_body,
        grid=(_NUM_CLASSES, 3),
        in_specs=[pl.BlockSpec((8, 1, 384, 128), lambda c, r: (0, c, r, 0))],
        out_specs=pl.BlockSpec((1, 384, 128), lambda c, r: (c, r, 0)),
        out_shape=jax.ShapeDtypeStruct((_NUM_CLASSES, 1152, 128), jnp.float32),
    )(x)


def _zero_vmem(ref, nwords):
    z = jnp.zeros((16,), jnp.int32)

    def z_body(i, _):
        ref[pl.ds(i * 16, 16)] = z
        return 0

    lax.fori_loop(0, nwords // 16, z_body, 0)


def _scan_top(hist_ref, nbins, base, target):
    iota = lax.iota(jnp.int32, 16)
    neg = jnp.int32(-2147483647)
    nch = nbins // 16

    def cond(carry):
        found, j, _, _, _ = carry
        return jnp.logical_and(jnp.logical_not(found), j < nch)

    def body(carry):
        _, j, b, cabove, cnt = carry
        c = nch - 1 - j
        h = hist_ref[pl.ds(c * 16, 16)]
        cum = plsc.cumsum(h)
        s = jnp.max(cum)
        ge = (cnt + s) - (cum - h)
        mask = ge >= target
        hit = (cnt + s) >= target
        l_ = jnp.max(jnp.where(mask, iota, -1))
        cum_at = jnp.max(jnp.where(iota == l_, cum, neg))
        nb = c * 16 + l_
        ncab = cnt + s - cum_at
        return (hit, j + 1, jnp.where(hit, nb, b),
                jnp.where(hit, ncab, cabove), cnt + s)

    res = lax.while_loop(cond, body, (jnp.bool_(False), jnp.int32(0),
                                      jnp.int32(-1), base, base))
    return res[0], res[2], res[3]


def _scan_bottom(hist_ref, nbins, target):
    iota = lax.iota(jnp.int32, 16)
    neg = jnp.int32(-2147483647)
    nch = nbins // 16

    def cond(carry):
        found, j, _, _, _ = carry
        return jnp.logical_and(jnp.logical_not(found), j < nch)

    def body(carry):
        _, j, b, cbelow, cnt = carry
        h = hist_ref[pl.ds(j * 16, 16)]
        cum = plsc.cumsum(h)
        s = jnp.max(cum)
        le = cnt + cum
        mask = le >= target
        hit = (cnt + s) >= target
        l_ = jnp.min(jnp.where(mask, iota, 99))
        cum_at = jnp.max(jnp.where(iota == l_, cum, neg))
        h_at = jnp.max(jnp.where(iota == l_, h, neg))
        nb = j * 16 + l_
        ncb = cnt + cum_at - h_at
        return (hit, j + 1, jnp.where(hit, nb, b),
                jnp.where(hit, ncb, cbelow), cnt + s)

    res = lax.while_loop(cond, body, (jnp.bool_(False), jnp.int32(0),
                                      jnp.int32(0), jnp.int32(0), jnp.int32(0)))
    return res[2], res[3]


def _sum_hist(hist_ref, nbins):
    def body(i, acc):
        return acc + hist_ref[pl.ds(i * 16, 16)]

    v = lax.fori_loop(0, nbins // 16, body, jnp.zeros((16,), jnp.int32))
    return jnp.sum(v)


def _merge_hist(hist_hbm_refs, hbuf, H):
    pltpu.sync_copy(hist_hbm_refs, hbuf)

    def m_body(i, _):
        H[pl.ds(i * 16, 16)] = hbuf[0, pl.ds(i * 16, 16)] + hbuf[1, pl.ds(i * 16, 16)]
        return 0

    lax.fori_loop(0, _HBINS // 16, m_body, 0)


_sc_mesh = plsc.VectorSubcoreMesh(core_axis_name="c", subcore_axis_name="s")


@functools.partial(
    pl.kernel,
    out_type=(
        jax.ShapeDtypeStruct((_NW * _CAPROW,), jnp.int32),
        jax.ShapeDtypeStruct((_NW * _CAPROW,), jnp.int32),
        jax.ShapeDtypeStruct((_NW * 128,), jnp.int32),
    ),
    mesh=_sc_mesh,
    compiler_params=pltpu.CompilerParams(needs_layout_passes=False),
    scratch_types=[
        pltpu.VMEM((_CHUNK,), jnp.float32),
        pltpu.VMEM((_HBINS,), jnp.int32),
        pltpu.VMEM((_CAPROW,), jnp.int32),
        pltpu.VMEM((_CAPROW,), jnp.int32),
        pltpu.VMEM((128,), jnp.int32),
        pltpu.SMEM((4,), jnp.int32),
    ],
)
def _sc_select(smf_hbm, ck_hbm, cf_hbm, cnt_hbm,
               vals, hist, ck, cf, cntv, off_ref):
    cid = lax.axis_index("c")
    sid = lax.axis_index("s")
    wid = sid * 2 + cid
    pltpu.sync_copy(smf_hbm.at[pl.ds(wid * _CHUNK, _CHUNK)], vals)
    _zero_vmem(hist, _HBINS)
    ones = jnp.ones((16,), jnp.int32)
    iota = lax.iota(jnp.int32, 16)

    def h_body(i, vacc):
        v = vals[pl.ds(i * 16, 16)]
        b = lax.bitcast_convert_type(v, jnp.int32)
        valid = v > _THRESH
        key = b - _BIAS
        bin_ = lax.shift_right_logical(key, 15)
        bin_ = jnp.where(valid, bin_, 0)
        plsc.addupdate_scatter(hist, [bin_], ones, mask=valid)
        return vacc + valid.astype(jnp.int32)

    vacc = lax.fori_loop(0, _NVEC, h_body, jnp.zeros((16,), jnp.int32),
                         unroll=4)
    n_valid_local = jnp.sum(vacc)
    found_l, L, _ = _scan_top(hist, _HBINS, jnp.int32(0), jnp.int32(_TOPK))
    thr = jnp.where(found_l, L * 32768, jnp.int32(-2147483647))
    off_ref[0] = 0

    j0 = wid * _CHUNK
    c0 = j0 // _N
    n0 = j0 - c0 * _N
    flat0 = n0 * _NUM_CLASSES + c0 + iota * _NUM_CLASSES
    tob0 = (_N - n0) // 16

    def s_body(i, carry):
        flatv, tob = carry
        v = vals[pl.ds(i * 16, 16)]
        b = lax.bitcast_convert_type(v, jnp.int32)
        valid = v > _THRESH
        key = b - _BIAS
        m = jnp.logical_and(valid, key >= thr)
        pc = plsc.all_reduce_population_count(m)[0]

        @pl.when(pc > 0)
        def _():
            off = off_ref[0]

            @pl.when(off < _CAP)
            def _():
                plsc.store_compressed(ck.at[pl.ds(off, 16)], key, mask=m)
                plsc.store_compressed(cf.at[pl.ds(off, 16)], flatv, mask=m)
                off_ref[0] = off + pc

        cross = tob == 1
        delta = jnp.where(cross, 240 - _TOT + 1, 240)
        tob2 = jnp.where(cross, jnp.int32(9216), tob - 1)
        return flatv + delta, tob2

    lax.fori_loop(0, _NVEC, s_body, (flat0, tob0), unroll=2)
    cnt_final = jnp.minimum(off_ref[0], _CAP)

    def c_body(i, _):
        val = jnp.where(i < 4, cnt_final, n_valid_local)
        cntv[pl.ds(i * 16, 16)] = jnp.broadcast_to(val, (16,))
        return 0

    lax.fori_loop(0, 8, c_body, 0)
    pltpu.sync_copy(ck, ck_hbm.at[pl.ds(wid * _CAPROW, _CAPROW)])
    pltpu.sync_copy(cf, cf_hbm.at[pl.ds(wid * _CAPROW, _CAPROW)])
    pltpu.sync_copy(cntv, cnt_hbm.at[pl.ds(wid * 128, 128)])


def _reg_body(x_ref, ck_ref, cf_ref, cnt_ref, o_ref, acc_ref, si_ref):
    r = pl.program_id(0)

    @pl.when(r == 0)
    def _():
        for i in range(12):
            acc_ref[i] = 0.0
        cnt_blk = cnt_ref[...]
        num_valid = jnp.sum(cnt_blk[:, 64:65])
        found = num_valid >= _TOPK
        kb = ck_ref[...]
        fb = cf_ref[...]
        cnts = cnt_blk[:, 0:1]
        pos = lax.broadcasted_iota(jnp.int32, (_NW, _CAPROW), 1)
        maskc = pos < cnts
        k_t = jnp.float32(_TOPK)

        def fcnt(x):
            return jnp.sum(jnp.where(
                jnp.logical_and(maskc, kb >= x), 1.0, 0.0))

        def v_body(_, lohi):
            lo, hi = lohi
            mid = lax.shift_right_logical(lo + hi + 1, 1)
            take = fcnt(mid) >= k_t
            return jnp.where(take, mid, lo), jnp.where(take, hi, mid - 1)

        lo0 = jnp.int32(0)
        hi0 = jnp.int32((1 << 26) - 1)
        v26, _ = lax.fori_loop(0, 26, v_body, (lo0, hi0))
        count_gt = fcnt(v26 + 1)
        rr = jnp.float32(_TOPK) - count_gt
        eqmask = jnp.logical_and(maskc, kb == v26)

        def gcnt(x):
            return jnp.sum(jnp.where(
                jnp.logical_and(eqmask, fb <= x), 1.0, 0.0))

        def t_body(_, lohi):
            lo, hi = lohi
            mid = lax.shift_right_logical(lo + hi, 1)
            take = gcnt(mid) >= rr
            return jnp.where(take, lo, mid + 1), jnp.where(take, mid, hi)

        istar, _ = lax.fori_loop(0, 22, t_body,
                                 (jnp.int32(0), jnp.int32((1 << 22) - 1)))
        thresh_key = jnp.int32(0x3D4CCCCD - _BIAS)
        si_ref[0] = jnp.where(found, v26, thresh_key)
        si_ref[1] = jnp.where(found, istar, jnp.int32(-1))
        si_ref[2] = jnp.where(num_valid > 0, 1, 0)

    v26s = si_ref[0]
    istar = si_ref[1]
    blk = x_ref[...]
    row2d = lax.broadcasted_iota(jnp.int32, (128, 128), 0)
    lane2d = lax.broadcasted_iota(jnp.int32, (128, 128), 1)
    n = (r * 128 + row2d) * 128 + lane2d
    keep = None
    for c in range(_NUM_CLASSES):
        key_c = lax.bitcast_convert_type(blk[c], jnp.int32) - _BIAS
        flat = n * _NUM_CLASSES + c
        sel = jnp.logical_or(
            key_c > v26s, jnp.logical_and(key_c == v26s, flat <= istar))
        keep = sel if keep is None else jnp.logical_or(keep, sel)
    x = blk[3]
    y = blk[5]
    lab = x > y
    fx = jnp.logical_and(keep, lab).astype(jnp.float32)
    fy = jnp.logical_and(keep, jnp.logical_not(lab)).astype(jnp.float32)
    acc_ref[0] += jnp.sum(fx)
    acc_ref[1] += jnp.sum(fx * x)
    acc_ref[2] += jnp.sum(fx * y)
    acc_ref[3] += jnp.sum(fx * x * x)
    acc_ref[4] += jnp.sum(fx * x * y)
    acc_ref[5] += jnp.sum(fy)
    acc_ref[6] += jnp.sum(fy * x)
    acc_ref[7] += jnp.sum(fy * y)
    acc_ref[8] += jnp.sum(fy * x * x)
    acc_ref[9] += jnp.sum(fy * x * y)

    @pl.when(r == 8)
    def _():
        nX = acc_ref[0]
        sxX, syX, sxxX, sxyX = acc_ref[1], acc_ref[2], acc_ref[3], acc_ref[4]
        nY = acc_ref[5]
        sxY, syY, sxxY, sxyY = acc_ref[6], acc_ref[7], acc_ref[8], acc_ref[9]
        slope_x = (sxyX - sxX * syX / nX) / (sxxX - sxX * sxX / nX)
        slope_y = (sxyY - sxY * syY / nY) / (sxxY - sxY * sxY / nY)
        t = jnp.abs((slope_y - slope_x) / (1.0 + slope_y * slope_x + 1e-05))
        tv = jnp.full((8, 128), t)
        hi = tv > 2.414213562373095
        mid = tv > 0.414213562373095
        yofs = jnp.where(hi, jnp.float32(1.5707963267948966),
                         jnp.where(mid, jnp.float32(0.7853981633974483), 0.0))
        z = jnp.where(hi, -1.0 / tv,
                      jnp.where(mid, (tv - 1.0) / (tv + 1.0), tv))
        z2 = z * z
        p = (((8.05374449538e-2 * z2 - 1.38776856032e-1) * z2
              + 1.99777106478e-1) * z2 - 3.33329491539e-1) * z2 * z + z
        ang = (yofs + p) * jnp.float32(57.29577951308232)
        cond = jnp.logical_and(si_ref[2] > 0, nX > 0.0)
        o_ref[...] = jnp.where(cond, ang, jnp.zeros((8, 128), jnp.float32))


def _tc_regression(sm_cm, ck, cf, cnt):
    out = pl.pallas_call(
        _reg_body,
        grid=(9,),
        in_specs=[
            pl.BlockSpec((_NUM_CLASSES, 128, 128), lambda r: (0, r, 0)),
            pl.BlockSpec((_NW, _CAPROW), lambda r: (0, 0)),
            pl.BlockSpec((_NW, _CAPROW), lambda r: (0, 0)),
            pl.BlockSpec((_NW, 128), lambda r: (0, 0)),
        ],
        out_specs=pl.BlockSpec((8, 128), lambda r: (0, 0)),
        out_shape=jax.ShapeDtypeStruct((8, 128), jnp.float32),
        scratch_shapes=[pltpu.SMEM((16,), jnp.float32),
                        pltpu.SMEM((8,), jnp.int32)],
    )(sm_cm, ck.reshape(_NW, _CAPROW), cf.reshape(_NW, _CAPROW),
      cnt.reshape(_NW, 128))
    return out[0, 0]


def kernel(cls_score):
    sm_cm = _scores_mean_cm(cls_score)
    smf = sm_cm.reshape(_TOT)
    ck, cf, cnt = _sc_select(smf)
    return _tc_regression(sm_cm, ck, cf, cnt).reshape(())


# speedup vs baseline: 1.2522x; 1.2522x over previous
import functools

import jax
import jax.numpy as jnp
from jax import lax
from jax.experimental import pallas as pl
from jax.experimental.pallas import tpu as pltpu
from jax.experimental.pallas import tpu_sc as plsc

_NUM_CLASSES = 15
_THRESH = 0.05
_TOPK = 2000
_N = 384 * 384
_TOT = _N * _NUM_CLASSES

_NW = 32
_CHUNK = _TOT // _NW
_NVEC = _CHUNK // 16
_HBINS = 2048
_L2BINS = 4096
_FBINS = 2048
_CAP = 1024
_BIAS = 0x3D000000
_BIG_FLAT = _TOT + 1


def _mean_body(x_ref, o_ref):
    s = jax.nn.sigmoid(x_ref[...])
    o_ref[...] = jnp.mean(s, axis=0)


def _scores_mean_cm(cls_score):
    x = cls_score.reshape(8, _NUM_CLASSES, 1152, 128)
    return pl.pallas_call(
        _mean_body,
        grid=(_NUM_CLASSES, 3),
        in_specs=[pl.BlockSpec((8, 1, 384, 128), lambda c, r: (0, c, r, 0))],
        out_specs=pl.BlockSpec((1, 384, 128), lambda c, r: (c, r, 0)),
        out_shape=jax.ShapeDtypeStruct((_NUM_CLASSES, 1152, 128), jnp.float32),
    )(x)


def _zero_vmem(ref, nwords):
    z = jnp.zeros((16,), jnp.int32)

    def z_body(i, _):
        ref[pl.ds(i * 16, 16)] = z
        return 0

    lax.fori_loop(0, nwords // 16, z_body, 0)


def _scan_top(hist_ref, nbins, base, target):
    iota = lax.iota(jnp.int32, 16)
    neg = jnp.int32(-2147483647)
    nch = nbins // 16

    def cond(carry):
        found, j, _, _, _ = carry
        return jnp.logical_and(jnp.logical_not(found), j < nch)

    def body(carry):
        _, j, b, cabove, cnt = carry
        c = nch - 1 - j
        h = hist_ref[pl.ds(c * 16, 16)]
        cum = plsc.cumsum(h)
        s = jnp.max(cum)
        ge = (cnt + s) - (cum - h)
        mask = ge >= target
        hit = (cnt + s) >= target
        l_ = jnp.max(jnp.where(mask, iota, -1))
        cum_at = jnp.max(jnp.where(iota == l_, cum, neg))
        nb = c * 16 + l_
        ncab = cnt + s - cum_at
        return (hit, j + 1, jnp.where(hit, nb, b),
                jnp.where(hit, ncab, cabove), cnt + s)

    res = lax.while_loop(cond, body, (jnp.bool_(False), jnp.int32(0),
                                      jnp.int32(-1), base, base))
    return res[0], res[2], res[3]


def _scan_bottom(hist_ref, nbins, target):
    iota = lax.iota(jnp.int32, 16)
    neg = jnp.int32(-2147483647)
    nch = nbins // 16

    def cond(carry):
        found, j, _, _, _ = carry
        return jnp.logical_and(jnp.logical_not(found), j < nch)

    def body(carry):
        _, j, b, cbelow, cnt = carry
        h = hist_ref[pl.ds(j * 16, 16)]
        cum = plsc.cumsum(h)
        s = jnp.max(cum)
        le = cnt + cum
        mask = le >= target
        hit = (cnt + s) >= target
        l_ = jnp.min(jnp.where(mask, iota, 99))
        cum_at = jnp.max(jnp.where(iota == l_, cum, neg))
        h_at = jnp.max(jnp.where(iota == l_, h, neg))
        nb = j * 16 + l_
        ncb = cnt + cum_at - h_at
        return (hit, j + 1, jnp.where(hit, nb, b),
                jnp.where(hit, ncb, cbelow), cnt + s)

    res = lax.while_loop(cond, body, (jnp.bool_(False), jnp.int32(0),
                                      jnp.int32(0), jnp.int32(0), jnp.int32(0)))
    return res[2], res[3]


def _sum_hist(hist_ref, nbins):
    def body(i, acc):
        return acc + hist_ref[pl.ds(i * 16, 16)]

    v = lax.fori_loop(0, nbins // 16, body, jnp.zeros((16,), jnp.int32))
    return jnp.sum(v)


def _merge_hist(hist_hbm_refs, hbuf, H):
    pltpu.sync_copy(hist_hbm_refs, hbuf)

    def m_body(i, _):
        H[pl.ds(i * 16, 16)] = hbuf[0, pl.ds(i * 16, 16)] + hbuf[1, pl.ds(i * 16, 16)]
        return 0

    lax.fori_loop(0, _HBINS // 16, m_body, 0)


_sc_mesh = plsc.VectorSubcoreMesh(core_axis_name="c", subcore_axis_name="s")


@functools.partial(
    pl.kernel,
    out_type=(
        jax.ShapeDtypeStruct((2, _HBINS), jnp.int32),
        jax.ShapeDtypeStruct((_NW * _NVEC,), jnp.int32),
    ),
    mesh=_sc_mesh,
    compiler_params=pltpu.CompilerParams(needs_layout_passes=False),
    scratch_types=[
        pltpu.VMEM((_CHUNK,), jnp.float32),
        pltpu.VMEM((_HBINS,), jnp.int32),
        pltpu.VMEM((_NVEC,), jnp.int32),
        pltpu.VMEM_SHARED((16, _HBINS), jnp.int32),
        pltpu.VMEM((16, _HBINS), jnp.int32),
    ],
)
def _sc_hist1(smf_hbm, out_hbm, gmax_hbm, vals, hist, gmax, shared, redbuf):
    cid = lax.axis_index("c")
    sid = lax.axis_index("s")
    wid = sid * 2 + cid
    pltpu.sync_copy(smf_hbm.at[pl.ds(wid * _CHUNK, _CHUNK)], vals)
    _zero_vmem(hist, _HBINS)
    ones = jnp.ones((16,), jnp.int32)
    iota = lax.iota(jnp.int32, 16)
    lane8 = jnp.bitwise_and(iota, 7)
    neg = jnp.full((16,), -2147483647, jnp.int32)

    def g_body(g, _):
        gmv = neg
        for t in range(16):
            i = g * 16 + t
            v = vals[pl.ds(i * 16, 16)]
            b = lax.bitcast_convert_type(v, jnp.int32)
            valid = v > _THRESH
            key = b - _BIAS
            bin_ = jnp.where(valid, lax.shift_right_logical(key, 15), 0)
            plsc.addupdate_scatter(hist, [bin_], ones, mask=valid)
            gmv = jnp.maximum(gmv, jnp.where(valid, key, neg))
        gmax[pl.ds(g * 16, 16)] = gmv
        return 0

    lax.fori_loop(0, _NVEC // 16, g_body, 0)
    pltpu.sync_copy(gmax, gmax_hbm.at[pl.ds(wid * _NVEC, _NVEC)])

    pltpu.sync_copy(hist, shared.at[sid])
    plsc.subcore_barrier()

    @pl.when(sid == 0)
    def _():
        pltpu.sync_copy(shared, redbuf)

        def red(i, _):
            acc = redbuf[0, pl.ds(i * 16, 16)]
            for t in range(1, 16):
                acc = acc + redbuf[t, pl.ds(i * 16, 16)]
            hist[pl.ds(i * 16, 16)] = acc
            return 0

        lax.fori_loop(0, _HBINS // 16, red, 0)
        pltpu.sync_copy(hist, out_hbm.at[cid])


@functools.partial(
    pl.kernel,
    out_type=(
        jax.ShapeDtypeStruct((_NW * _CAP,), jnp.int32),
        jax.ShapeDtypeStruct((_NW * _CAP,), jnp.int32),
        jax.ShapeDtypeStruct((_NW * 128,), jnp.int32),
        jax.ShapeDtypeStruct((16,), jnp.int32),
    ),
    mesh=_sc_mesh,
    compiler_params=pltpu.CompilerParams(needs_layout_passes=False),
    scratch_types=[
        pltpu.VMEM((_CHUNK,), jnp.float32),
        pltpu.VMEM((2, _HBINS), jnp.int32),
        pltpu.VMEM((_HBINS,), jnp.int32),
        pltpu.VMEM((_NVEC,), jnp.int32),
        pltpu.VMEM((_CAP,), jnp.int32),
        pltpu.VMEM((_CAP,), jnp.int32),
        pltpu.VMEM((128,), jnp.int32),
        pltpu.VMEM((16,), jnp.int32),
        pltpu.SMEM((4,), jnp.int32),
    ],
)
def _sc_compact(smf_hbm, gmax_hbm, hist_hbm, ck_hbm, cf_hbm, cnt_hbm, mb_hbm,
                vals, hbuf, H, gmax, ck, cf, cntv, mbv, off_ref):
    cid = lax.axis_index("c")
    sid = lax.axis_index("s")
    wid = sid * 2 + cid
    pltpu.sync_copy(smf_hbm.at[pl.ds(wid * _CHUNK, _CHUNK)], vals)
    pltpu.sync_copy(gmax_hbm.at[pl.ds(wid * _NVEC, _NVEC)], gmax)
    _merge_hist(hist_hbm, hbuf, H)
    found, b1, cabove1 = _scan_top(H, _HBINS, jnp.int32(0), jnp.int32(_TOPK))
    b1 = jnp.where(found, b1, jnp.int32(-1))
    thr_lo = b1 * 32768
    iota = lax.iota(jnp.int32, 16)
    off_ref[0] = 0

    def g_body(g, _):
        gv = gmax[pl.ds(g * 16, 16)]
        m0 = gv >= thr_lo

        @pl.when(jnp.any(m0))
        def _():
            def col_cond(carry):
                m, _ = carry
                return jnp.any(m > 0)

            def col_body(carry):
                m, off = carry
                l = plsc.all_reduce_ffs(m > 0)[0]
                idx = g * 256 + l + iota * 16
                vv = plsc.load_gather(vals, [idx])
                b = lax.bitcast_convert_type(vv, jnp.int32)
                key = b - _BIAS
                mm = jnp.logical_and(
                    vv > _THRESH,
                    lax.shift_right_logical(key, 15) == b1)
                incl = plsc.cumsum(mm.astype(jnp.int32))
                tot = jnp.max(incl)
                jv = wid * _CHUNK + idx
                cdiv = jv // _N
                flat = (jv - cdiv * _N) * _NUM_CLASSES + cdiv
                pos = off + incl - 1
                m2 = jnp.logical_and(mm, pos < _CAP)
                pos = jnp.where(m2, pos, 0)
                plsc.store_scatter(ck, [pos], key, mask=m2)
                plsc.store_scatter(cf, [pos], flat, mask=m2)
                return jnp.where(iota != l, m, 0), off + tot

            m_end, off_end = lax.while_loop(
                col_cond, col_body, (m0.astype(jnp.int32), off_ref[0]))
            off_ref[0] = off_end

        return 0

    lax.fori_loop(0, _NVEC // 16, g_body, 0)
    cnt_final = jnp.minimum(off_ref[0], _CAP)

    def c_body(i, _):
        cntv[pl.ds(i * 16, 16)] = jnp.broadcast_to(cnt_final, (16,))
        return 0

    lax.fori_loop(0, 8, c_body, 0)
    pltpu.sync_copy(ck, ck_hbm.at[pl.ds(wid * _CAP, _CAP)])
    pltpu.sync_copy(cf, cf_hbm.at[pl.ds(wid * _CAP, _CAP)])
    pltpu.sync_copy(cntv, cnt_hbm.at[pl.ds(wid * 128, 128)])

    @pl.when(wid == 0)
    def _():
        num_valid = _sum_hist(H, _HBINS)
        foundi = jnp.int32(1) - jnp.where(found, 0, 1)
        is0 = (iota == 0).astype(jnp.int32)
        is1 = (iota == 1).astype(jnp.int32)
        is2 = (iota == 2).astype(jnp.int32)
        is3 = (iota == 3).astype(jnp.int32)
        mbv[...] = (is0 * foundi + is1 * jnp.maximum(b1, 0)
                    + is2 * cabove1 + is3 * num_valid)
        pltpu.sync_copy(mbv, mb_hbm)


def _reg_body(mb_ref, x_ref, ck_ref, cf_ref, cnt_ref, o_ref, acc_ref, si_ref):
    r = pl.program_id(0)

    @pl.when(r == 0)
    def _():
        for i in range(12):
            acc_ref[i] = 0.0
        found = mb_ref[0] > 0
        b1 = mb_ref[1]
        cabove1 = mb_ref[2]
        num_valid = mb_ref[3]
        kb = ck_ref[...]
        fb = cf_ref[...]
        cnts = cnt_ref[...][:, 0:1]
        pos = lax.broadcasted_iota(jnp.int32, (_NW, _CAP), 1)
        maskc = pos < cnts
        k_t = jnp.float32(_TOPK) - cabove1.astype(jnp.float32)

        def fcnt(x):
            return jnp.sum(jnp.where(
                jnp.logical_and(maskc, kb >= x), 1.0, 0.0))

        def v_body(_, lohi):
            lo, hi = lohi
            mid = lax.shift_right_logical(lo + hi + 1, 1)
            take = fcnt(mid) >= k_t
            return jnp.where(take, mid, lo), jnp.where(take, hi, mid - 1)

        lo0 = b1 * 32768
        hi0 = lo0 + 32767
        v26, _ = lax.fori_loop(0, 15, v_body, (lo0, hi0))
        count_gt = cabove1.astype(jnp.float32) + fcnt(v26 + 1)
        rr = jnp.float32(_TOPK) - count_gt
        eqmask = jnp.logical_and(maskc, kb == v26)

        def gcnt(x):
            return jnp.sum(jnp.where(
                jnp.logical_and(eqmask, fb <= x), 1.0, 0.0))

        def t_body(_, lohi):
            lo, hi = lohi
            mid = lax.shift_right_logical(lo + hi, 1)
            take = gcnt(mid) >= rr
            return jnp.where(take, lo, mid + 1), jnp.where(take, mid, hi)

        istar, _ = lax.fori_loop(0, 22, t_body,
                                 (jnp.int32(0), jnp.int32((1 << 22) - 1)))
        thresh_key = jnp.int32(0x3D4CCCCD - _BIAS)
        si_ref[0] = jnp.where(found, v26, thresh_key)
        si_ref[1] = jnp.where(found, istar, jnp.int32(-1))
        si_ref[2] = (num_valid > 0).astype(jnp.int32)

    v26s = si_ref[0]
    istar = si_ref[1]
    blk = x_ref[...]
    row2d = lax.broadcasted_iota(jnp.int32, (128, 128), 0)
    lane2d = lax.broadcasted_iota(jnp.int32, (128, 128), 1)
    n = (r * 128 + row2d) * 128 + lane2d
    keep = None
    for c in range(_NUM_CLASSES):
        key_c = lax.bitcast_convert_type(blk[c], jnp.int32) - _BIAS
        flat = n * _NUM_CLASSES + c
        sel = jnp.logical_or(
            key_c > v26s, jnp.logical_and(key_c == v26s, flat <= istar))
        keep = sel if keep is None else jnp.logical_or(keep, sel)
    x = blk[3]
    y = blk[5]
    lab = x > y
    fx = jnp.logical_and(keep, lab).astype(jnp.float32)
    fy = jnp.logical_and(keep, jnp.logical_not(lab)).astype(jnp.float32)
    acc_ref[0] += jnp.sum(fx)
    acc_ref[1] += jnp.sum(fx * x)
    acc_ref[2] += jnp.sum(fx * y)
    acc_ref[3] += jnp.sum(fx * x * x)
    acc_ref[4] += jnp.sum(fx * x * y)
    acc_ref[5] += jnp.sum(fy)
    acc_ref[6] += jnp.sum(fy * x)
    acc_ref[7] += jnp.sum(fy * y)
    acc_ref[8] += jnp.sum(fy * x * x)
    acc_ref[9] += jnp.sum(fy * x * y)

    @pl.when(r == 8)
    def _():
        nX = acc_ref[0]
        sxX, syX, sxxX, sxyX = acc_ref[1], acc_ref[2], acc_ref[3], acc_ref[4]
        nY = acc_ref[5]
        sxY, syY, sxxY, sxyY = acc_ref[6], acc_ref[7], acc_ref[8], acc_ref[9]
        slope_x = (sxyX - sxX * syX / nX) / (sxxX - sxX * sxX / nX)
        slope_y = (sxyY - sxY * syY / nY) / (sxxY - sxY * sxY / nY)
        t = jnp.abs((slope_y - slope_x) / (1.0 + slope_y * slope_x + 1e-05))
        tv = jnp.full((8, 128), t)
        hi = tv > 2.414213562373095
        mid = tv > 0.414213562373095
        yofs = jnp.where(hi, jnp.float32(1.5707963267948966),
                         jnp.where(mid, jnp.float32(0.7853981633974483), 0.0))
        z = jnp.where(hi, -1.0 / tv,
                      jnp.where(mid, (tv - 1.0) / (tv + 1.0), tv))
        z2 = z * z
        p = (((8.05374449538e-2 * z2 - 1.38776856032e-1) * z2
              + 1.99777106478e-1) * z2 - 3.33329491539e-1) * z2 * z + z
        ang = (yofs + p) * jnp.float32(57.29577951308232)
        cond = jnp.logical_and(si_ref[2] > 0, nX > 0.0)
        o_ref[...] = jnp.where(cond, ang, jnp.zeros((8, 128), jnp.float32))


def _tc_regression(sm_cm, meta_b, ck, cf, cnt):
    out = pl.pallas_call(
        _reg_body,
        grid=(9,),
        in_specs=[
            pl.BlockSpec(memory_space=pltpu.SMEM),
            pl.BlockSpec((_NUM_CLASSES, 128, 128), lambda r: (0, r, 0)),
            pl.BlockSpec((_NW, _CAP), lambda r: (0, 0)),
            pl.BlockSpec((_NW, _CAP), lambda r: (0, 0)),
            pl.BlockSpec((_NW, 128), lambda r: (0, 0)),
        ],
        out_specs=pl.BlockSpec((8, 128), lambda r: (0, 0)),
        out_shape=jax.ShapeDtypeStruct((8, 128), jnp.float32),
        scratch_shapes=[pltpu.SMEM((16,), jnp.float32),
                        pltpu.SMEM((8,), jnp.int32)],
    )(meta_b, sm_cm, ck.reshape(_NW, _CAP), cf.reshape(_NW, _CAP),
      cnt.reshape(_NW, 128))
    return out[0, 0]


def kernel(cls_score):
    sm_cm = _scores_mean_cm(cls_score)
    smf = sm_cm.reshape(_TOT)
    hist, gmax = _sc_hist1(smf)
    ck, cf, cnt, meta_b = _sc_compact(smf, gmax, hist)
    return _tc_regression(sm_cm, meta_b, ck, cf, cnt).reshape(())
